# baseline (speedup 1.0000x reference)
import jax
import jax.numpy as jnp
import numpy as np
from jax.experimental import pallas as pl
from jax.experimental.pallas import tpu as pltpu

_FACTOR = 8

_rng = np.random.RandomState(0)
_FIXED_IDX = np.stack(
    [_rng.choice(_FACTOR, _FACTOR, replace=False),
     _rng.choice(_FACTOR, _FACTOR, replace=False)],
).astype(np.int32)

_N_CHUNKS = 8
_N_BUFS = 8


def _body(idx_ref, img_ref, out_img_ref, out_idx_ref,
          bufs, in_sems, out_sems):
    c = img_ref.shape[0]
    cpc = c // _N_CHUNKS

    def in_copy(i):
        return pltpu.make_async_copy(
            img_ref.at[pl.ds(i * cpc, cpc)], bufs.at[i % _N_BUFS],
            in_sems.at[i % _N_BUFS])

    def out_copy(i):
        return pltpu.make_async_copy(
            bufs.at[i % _N_BUFS], out_img_ref.at[pl.ds(i * cpc, cpc)],
            out_sems.at[i % _N_BUFS])

    for i in range(2):
        in_copy(i).start()

    nz = idx_ref[0, 0] != 0
    for i in range(2):
        for j in range(_FACTOR):
            if (i, j) != (0, 0):
                nz = nz | (idx_ref[i, j] != 0)
    for i in range(2):
        for j in range(_FACTOR):
            out_idx_ref[i, j] = jnp.where(
                nz, idx_ref[i, j], jnp.int32(_FIXED_IDX[i, j]))

    for i in range(_N_CHUNKS):
        in_copy(i).wait()
        out_copy(i).start()
        if i + 2 < _N_CHUNKS:
            in_copy(i + 2).start()
    for i in range(_N_CHUNKS):
        out_copy(i).wait()


def kernel(img, indices):
    c, h, w = img.shape
    cpc = c // _N_CHUNKS

    return pl.pallas_call(
        _body,
        in_specs=[
            pl.BlockSpec(memory_space=pltpu.SMEM),
            pl.BlockSpec(memory_space=pltpu.MemorySpace.HBM),
        ],
        out_specs=[
            pl.BlockSpec(memory_space=pltpu.MemorySpace.HBM),
            pl.BlockSpec(memory_space=pltpu.SMEM),
        ],
        out_shape=[
            jax.ShapeDtypeStruct((c, h, w), img.dtype),
            jax.ShapeDtypeStruct((2, _FACTOR), jnp.int32),
        ],
        scratch_shapes=[
            pltpu.VMEM((_N_BUFS, cpc, h, w), img.dtype),
            pltpu.SemaphoreType.DMA((_N_BUFS,)),
            pltpu.SemaphoreType.DMA((_N_BUFS,)),
        ],
    )(indices, img)


# speedup vs baseline: 1.0733x; 1.0733x over previous
import jax
import jax.numpy as jnp
import numpy as np
from jax.experimental import pallas as pl
from jax.experimental.pallas import tpu as pltpu

_FACTOR = 8

_rng = np.random.RandomState(0)
_FIXED_IDX = np.stack(
    [_rng.choice(_FACTOR, _FACTOR, replace=False),
     _rng.choice(_FACTOR, _FACTOR, replace=False)],
).astype(np.int32)

_C_BLOCK = 26


def _body(idx_ref, img_ref, out_img_ref, out_idx_ref):
    out_img_ref[...] = img_ref[...]

    @pl.when(pl.program_id(0) == 0)
    def _():
        nz = idx_ref[0, 0] != 0
        for i in range(2):
            for j in range(_FACTOR):
                if (i, j) != (0, 0):
                    nz = nz | (idx_ref[i, j] != 0)
        for i in range(2):
            for j in range(_FACTOR):
                out_idx_ref[i, j] = jnp.where(
                    nz, idx_ref[i, j], jnp.int32(_FIXED_IDX[i, j]))


def kernel(img, indices):
    c, h, w = img.shape

    return pl.pallas_call(
        _body,
        grid=(pl.cdiv(c, _C_BLOCK),),
        in_specs=[
            pl.BlockSpec(memory_space=pltpu.SMEM),
            pl.BlockSpec((_C_BLOCK, h, w), lambda i: (i, 0, 0)),
        ],
        out_specs=[
            pl.BlockSpec((_C_BLOCK, h, w), lambda i: (i, 0, 0)),
            pl.BlockSpec(memory_space=pltpu.SMEM),
        ],
        out_shape=[
            jax.ShapeDtypeStruct((c, h, w), img.dtype),
            jax.ShapeDtypeStruct((2, _FACTOR), jnp.int32),
        ],
    )(indices, img)


# speedup vs baseline: 1.0740x; 1.0007x over previous
import jax
import jax.numpy as jnp
import numpy as np
from jax.experimental import pallas as pl
from jax.experimental.pallas import tpu as pltpu

_FACTOR = 8

_rng = np.random.RandomState(0)
_FIXED_IDX = np.stack(
    [_rng.choice(_FACTOR, _FACTOR, replace=False),
     _rng.choice(_FACTOR, _FACTOR, replace=False)],
).astype(np.int32)

_C_BLOCK = 27


def _body(idx_ref, img_ref, out_img_ref, out_idx_ref):
    out_img_ref[...] = img_ref[...]

    @pl.when(pl.program_id(0) == 0)
    def _():
        nz = idx_ref[0, 0] != 0
        for i in range(2):
            for j in range(_FACTOR):
                if (i, j) != (0, 0):
                    nz = nz | (idx_ref[i, j] != 0)
        for i in range(2):
            for j in range(_FACTOR):
                out_idx_ref[i, j] = jnp.where(
                    nz, idx_ref[i, j], jnp.int32(_FIXED_IDX[i, j]))


def kernel(img, indices):
    c, h, w = img.shape

    return pl.pallas_call(
        _body,
        grid=(pl.cdiv(c, _C_BLOCK),),
        in_specs=[
            pl.BlockSpec(memory_space=pltpu.SMEM),
            pl.BlockSpec((_C_BLOCK, h, w), lambda i: (i, 0, 0)),
        ],
        out_specs=[
            pl.BlockSpec((_C_BLOCK, h, w), lambda i: (i, 0, 0)),
            pl.BlockSpec(memory_space=pltpu.SMEM),
        ],
        compiler_params=pltpu.CompilerParams(
            vmem_limit_bytes=64 * 1024 * 1024),
        out_shape=[
            jax.ShapeDtypeStruct((c, h, w), img.dtype),
            jax.ShapeDtypeStruct((2, _FACTOR), jnp.int32),
        ],
    )(indices, img)
